# baseline (speedup 1.0000x reference)
import functools
import jax
import jax.numpy as jnp
from jax import lax
from jax.experimental import pallas as pl
from jax.experimental.pallas import tpu as pltpu
from jax.experimental.pallas import tpu_sc as plsc

B, C, H, W = 8, 19, 512, 512
NC, NS, L = 2, 16, 16
NW = NC * NS
TCB = 5
SCB = B - TCB
RH = 8
CW = 256
GBLK = SCB * H // RH
TPW = GBLK // NW
NBLK = TPW * (W // CW)
NACC = 4
BH = 64

_mesh = plsc.VectorSubcoreMesh(core_axis_name="c", subcore_axis_name="s")


@functools.partial(
    pl.kernel,
    out_type=jax.ShapeDtypeStruct((NW, 3, L), jnp.float32),
    mesh=_mesh,
    scratch_types=[
        pltpu.VMEM((C, RH, CW), jnp.float32),
        pltpu.VMEM((C, RH, CW), jnp.float32),
        pltpu.VMEM((RH, CW), jnp.int32),
        pltpu.VMEM((RH, CW), jnp.int32),
        pltpu.VMEM((3, L), jnp.float32),
        pltpu.SemaphoreType.DMA,
        pltpu.SemaphoreType.DMA,
        pltpu.SemaphoreType.DMA,
        pltpu.SemaphoreType.DMA,
    ],
    compiler_params=pltpu.CompilerParams(use_tc_tiling_on_sc=True),
)
def _objectosphere_sc(logits_hbm, sem_hbm, out_hbm,
                      buf0, buf1, sbuf0, sbuf1, acc,
                      sl0, sl1, ss0, ss1):
    w = lax.axis_index("s") * NC + lax.axis_index("c")
    zero = jnp.zeros((L,), jnp.float32)
    bufs = (buf0, buf1)
    sbufs = (sbuf0, sbuf1)
    sls = (sl0, sl1)
    sss = (ss0, ss1)

    def start(i, p):
        g = w + (i // 2) * NW
        b = TCB + g // (H // RH)
        r = (g % (H // RH)) * RH
        col = (i % 2) * CW
        pltpu.async_copy(
            logits_hbm.at[b, :, pl.ds(r, RH), pl.ds(col, CW)], bufs[p],
            sls[p])
        pltpu.async_copy(
            sem_hbm.at[b, pl.ds(r, RH), pl.ds(col, CW)], sbufs[p], sss[p])

    def wait(p):
        pltpu.make_async_copy(
            logits_hbm.at[0, :, pl.ds(0, RH), pl.ds(0, CW)], bufs[p],
            sls[p]).wait()
        pltpu.make_async_copy(
            sem_hbm.at[0, pl.ds(0, RH), pl.ds(0, CW)], sbufs[p],
            sss[p]).wait()

    def compute_block(p, carry):
        buf, sbuf = bufs[p], sbufs[p]

        def inner(j, c2):
            sus, sks, cus = c2
            sus, sks, cus = list(sus), list(sks), list(cus)
            s16 = pl.ds(j * L, L)
            for r in range(RH):
                a = r % NACC
                n = zero
                for c in range(C):
                    v = buf[c, r, s16]
                    n = n + v * v
                m = sbuf[r, s16] == 0
                sus[a] = sus[a] + jnp.where(m, n, 0.0)
                sks[a] = sks[a] + jnp.where(m, 0.0,
                                            jnp.maximum(1.0 - n, 0.0))
                cus[a] = cus[a] + jnp.where(m, 1.0, 0.0)
            return tuple(sus), tuple(sks), tuple(cus)

        zz = (zero,) * NACC
        sus, sks, cus = lax.fori_loop(0, CW // L, inner, (zz, zz, zz))
        su, sk, cu = carry
        return (su + sum(sus), sk + sum(sks), cu + sum(cus))

    start(0, 0)
    start(1, 1)

    def pair(t, carry):
        i0 = t * 2
        wait(0)
        carry = compute_block(0, carry)

        @pl.when(i0 + 2 < NBLK)
        def _():
            start(i0 + 2, 0)

        wait(1)
        carry = compute_block(1, carry)

        @pl.when(i0 + 3 < NBLK)
        def _():
            start(i0 + 3, 1)

        return carry

    su, sk, cu = lax.fori_loop(0, NBLK // 2, pair, (zero, zero, zero))
    acc[0, :] = su
    acc[1, :] = sk
    acc[2, :] = cu
    pltpu.sync_copy(acc, out_hbm.at[w])


def _tc_body(logits_ref, sem_ref, out_ref, acc):
    i = pl.program_id(0)
    j = pl.program_id(1)

    @pl.when((i == 0) & (j == 0))
    def _():
        acc[0] = 0.0
        acc[1] = 0.0
        acc[2] = 0.0

    x = logits_ref[0]
    n = jnp.sum(x * x, axis=0)
    m = sem_ref[0] == 0
    acc[0] += jnp.sum(jnp.where(m, n, 0.0))
    acc[1] += jnp.sum(jnp.where(m, 0.0, jnp.maximum(1.0 - n, 0.0)))
    acc[2] += jnp.sum(jnp.where(m, 1.0, 0.0))

    @pl.when((i == TCB - 1) & (j == H // BH - 1))
    def _():
        out_ref[0] = acc[0]
        out_ref[1] = acc[1]
        out_ref[2] = acc[2]


_tc_partials = pl.pallas_call(
    _tc_body,
    grid=(TCB, H // BH),
    in_specs=[
        pl.BlockSpec((1, C, BH, W), lambda i, j: (i, 0, j, 0)),
        pl.BlockSpec((1, BH, W), lambda i, j: (i, j, 0)),
    ],
    out_specs=pl.BlockSpec(memory_space=pltpu.SMEM),
    out_shape=jax.ShapeDtypeStruct((3,), jnp.float32),
    scratch_shapes=[pltpu.SMEM((3,), jnp.float32)],
)


def kernel(logits, sem_gt):
    sem32 = sem_gt.astype(jnp.int32)
    parts_sc = _objectosphere_sc(logits, sem32)
    parts_tc = _tc_partials(logits, sem32)
    sum_unk = jnp.sum(parts_sc[:, 0, :]) + parts_tc[0]
    sum_kn = jnp.sum(parts_sc[:, 1, :]) + parts_tc[1]
    n_unk = jnp.sum(parts_sc[:, 2, :]) + parts_tc[2]
    n_kn = jnp.float32(B * H * W) - n_unk
    loss_unk = jnp.where(n_unk > 0, sum_unk / jnp.maximum(n_unk, 1.0), 0.0)
    loss_kn = jnp.where(n_kn > 0, sum_kn / jnp.maximum(n_kn, 1.0), 0.0)
    return 10.0 * loss_unk + loss_kn


# speedup vs baseline: 1.0444x; 1.0444x over previous
import functools
import jax
import jax.numpy as jnp
from jax import lax
from jax.experimental import pallas as pl
from jax.experimental.pallas import tpu as pltpu
from jax.experimental.pallas import tpu_sc as plsc

B, C, H, W = 8, 19, 512, 512
NC, NS, L = 2, 16, 16
NW = NC * NS
TCB = 5
SCB = B - TCB
RH = 8
CW = 256
GBLK = SCB * H // RH
TPW = GBLK // NW
NBLK = TPW * (W // CW)
NACC = 4
BH = 128
BPB = H // BH
TBANDS = TCB * BPB

_mesh = plsc.VectorSubcoreMesh(core_axis_name="c", subcore_axis_name="s")


@functools.partial(
    pl.kernel,
    out_type=jax.ShapeDtypeStruct((NW, 3, L), jnp.float32),
    mesh=_mesh,
    scratch_types=[
        pltpu.VMEM((C, RH, CW), jnp.float32),
        pltpu.VMEM((C, RH, CW), jnp.float32),
        pltpu.VMEM((RH, CW), jnp.int32),
        pltpu.VMEM((RH, CW), jnp.int32),
        pltpu.VMEM((3, L), jnp.float32),
        pltpu.SemaphoreType.DMA,
        pltpu.SemaphoreType.DMA,
        pltpu.SemaphoreType.DMA,
        pltpu.SemaphoreType.DMA,
    ],
    compiler_params=pltpu.CompilerParams(use_tc_tiling_on_sc=True,
                                         skip_device_barrier=True),
)
def _objectosphere_sc(logits_hbm, sem_hbm, out_hbm,
                      buf0, buf1, sbuf0, sbuf1, acc,
                      sl0, sl1, ss0, ss1):
    w = lax.axis_index("s") * NC + lax.axis_index("c")
    zero = jnp.zeros((L,), jnp.float32)
    bufs = (buf0, buf1)
    sbufs = (sbuf0, sbuf1)
    sls = (sl0, sl1)
    sss = (ss0, ss1)

    def start(i, p):
        g = w + (i // 2) * NW
        b = TCB + g // (H // RH)
        r = (g % (H // RH)) * RH
        col = (i % 2) * CW
        pltpu.async_copy(
            logits_hbm.at[b, :, pl.ds(r, RH), pl.ds(col, CW)], bufs[p],
            sls[p])
        pltpu.async_copy(
            sem_hbm.at[b, pl.ds(r, RH), pl.ds(col, CW)], sbufs[p], sss[p])

    def wait(p):
        pltpu.make_async_copy(
            logits_hbm.at[0, :, pl.ds(0, RH), pl.ds(0, CW)], bufs[p],
            sls[p]).wait()
        pltpu.make_async_copy(
            sem_hbm.at[0, pl.ds(0, RH), pl.ds(0, CW)], sbufs[p],
            sss[p]).wait()

    def compute_block(p, carry):
        buf, sbuf = bufs[p], sbufs[p]

        def inner(j, c2):
            sus, sks, cus = c2
            sus, sks, cus = list(sus), list(sks), list(cus)
            s16 = pl.ds(j * L, L)
            for r in range(RH):
                a = r % NACC
                n = zero
                for c in range(C):
                    v = buf[c, r, s16]
                    n = n + v * v
                m = sbuf[r, s16] == 0
                sus[a] = sus[a] + jnp.where(m, n, 0.0)
                sks[a] = sks[a] + jnp.where(m, 0.0,
                                            jnp.maximum(1.0 - n, 0.0))
                cus[a] = cus[a] + jnp.where(m, 1.0, 0.0)
            return tuple(sus), tuple(sks), tuple(cus)

        zz = (zero,) * NACC
        sus, sks, cus = lax.fori_loop(0, CW // L, inner, (zz, zz, zz))
        su, sk, cu = carry
        return (su + sum(sus), sk + sum(sks), cu + sum(cus))

    start(0, 0)
    start(1, 1)

    def pair(t, carry):
        i0 = t * 2
        wait(0)
        carry = compute_block(0, carry)

        @pl.when(i0 + 2 < NBLK)
        def _():
            start(i0 + 2, 0)

        wait(1)
        carry = compute_block(1, carry)

        @pl.when(i0 + 3 < NBLK)
        def _():
            start(i0 + 3, 1)

        return carry

    su, sk, cu = lax.fori_loop(0, NBLK // 2, pair, (zero, zero, zero))
    acc[0, :] = su
    acc[1, :] = sk
    acc[2, :] = cu
    pltpu.sync_copy(acc, out_hbm.at[w])


def _fold(v):
    v = v.reshape(BH, W // 128, 128).sum(axis=1)
    return v.reshape(BH // 8, 8, 128).sum(axis=0)


def _tc_body(logits_ref, sem_ref, out_ref, acc):
    t = pl.program_id(0)

    @pl.when(t == 0)
    def _():
        acc[...] = jnp.zeros_like(acc)

    x = logits_ref[0]
    n = jnp.sum(x * x, axis=0)
    m = sem_ref[0] == 0
    acc[0] += _fold(jnp.where(m, n, 0.0))
    acc[1] += _fold(jnp.where(m, 0.0, jnp.maximum(1.0 - n, 0.0)))
    acc[2] += _fold(jnp.where(m, 1.0, 0.0))

    @pl.when(t == TBANDS - 1)
    def _():
        out_ref[0] = jnp.sum(acc[0])
        out_ref[1] = jnp.sum(acc[1])
        out_ref[2] = jnp.sum(acc[2])


_tc_partials = pl.pallas_call(
    _tc_body,
    grid=(TBANDS,),
    in_specs=[
        pl.BlockSpec((1, C, BH, W), lambda t: (t // BPB, 0, t % BPB, 0)),
        pl.BlockSpec((1, BH, W), lambda t: (t // BPB, t % BPB, 0)),
    ],
    out_specs=pl.BlockSpec(memory_space=pltpu.SMEM),
    out_shape=jax.ShapeDtypeStruct((3,), jnp.float32),
    scratch_shapes=[pltpu.VMEM((3, 8, 128), jnp.float32)],
)


def kernel(logits, sem_gt):
    sem32 = sem_gt.astype(jnp.int32)
    parts_sc = _objectosphere_sc(logits, sem32)
    parts_tc = _tc_partials(logits, sem32)
    sum_unk = jnp.sum(parts_sc[:, 0, :]) + parts_tc[0]
    sum_kn = jnp.sum(parts_sc[:, 1, :]) + parts_tc[1]
    n_unk = jnp.sum(parts_sc[:, 2, :]) + parts_tc[2]
    n_kn = jnp.float32(B * H * W) - n_unk
    loss_unk = jnp.where(n_unk > 0, sum_unk / jnp.maximum(n_unk, 1.0), 0.0)
    loss_kn = jnp.where(n_kn > 0, sum_kn / jnp.maximum(n_kn, 1.0), 0.0)
    return 10.0 * loss_unk + loss_kn
